# baseline (speedup 1.0000x reference)
import functools
import jax
import jax.numpy as jnp
from jax import lax
from jax.experimental import pallas as pl
from jax.experimental.pallas import tpu as pltpu
from jax.experimental.pallas import tpu_sc as plsc

VOCAB_ROWS = 1000000
D = 64
NC = 2
NS = 16
NW = NC * NS
NBLK = VOCAB_ROWS // 128
NMAIN = (NBLK // NW) * NW
CHUNK = 128

_params = pltpu.CompilerParams(
    use_tc_tiling_on_sc=True, needs_layout_passes=False,
    disable_bounds_checks=True)
_mesh = plsc.VectorSubcoreMesh(core_axis_name="c", subcore_axis_name="s")


def _iota16():
    return lax.iota(jnp.int32, 16)


def _relayout_body(wt_hbm, wtail_hbm, wlin_hbm, tbuf, rbuf, isem, osem):
    wid = lax.axis_index("s") * NC + lax.axis_index("c")
    iota = _iota16()
    zero = iota - iota
    cvecs = [iota + 16 * j for j in range(4)]

    def load(slot, j):
        return pltpu.make_async_copy(
            wt_hbm.at[pl.ds(0, 64),
                      pl.ds(pl.multiple_of((wid + NW * j) * 128, 128), 128)],
            tbuf.at[slot], isem.at[slot])

    def store(slot, j):
        return pltpu.make_async_copy(
            rbuf.at[slot],
            wlin_hbm.at[pl.ds(pl.multiple_of((wid + NW * j) * 128, 128), 128)],
            osem.at[slot])

    def transpose(slot):
        tb = tbuf.at[slot]
        rb = rbuf.at[slot]

        @plsc.parallel_loop(0, 128, unroll=4)
        def _rows(r):
            rsplat = zero + r
            for j in range(4):
                vals = plsc.load_gather(tb, [cvecs[j], rsplat])
                rb[r, pl.ds(16 * j, 16)] = vals

    NB = NMAIN // NW
    NQ = NB // 4

    for par in range(4):
        load(par, par).start()

    def body(m, _):
        for par in range(4):
            j = 4 * m + par
            load(par, j).wait()

            @pl.when(m > 0)
            def _drain():
                store(par, j - 4).wait()

            transpose(par)

            @pl.when(m < NQ - 1)
            def _next():
                load(par, j + 4).start()

            store(par, j).start()
        return _

    lax.fori_loop(0, NQ, body, None)
    for par in range(4):
        store(par, NB - 4 + par).wait()

    @pl.when(wid < 4)
    def _tail():
        off = pl.multiple_of((NMAIN + wid) * 128, 128)
        pltpu.make_async_copy(
            wt_hbm.at[pl.ds(0, 64), pl.ds(off, 128)], tbuf.at[0],
            isem.at[0]).start()
        pltpu.make_async_copy(
            wt_hbm.at[pl.ds(0, 64), pl.ds(off, 128)], tbuf.at[0],
            isem.at[0]).wait()
        transpose(0)
        pltpu.make_async_copy(
            rbuf.at[0], wlin_hbm.at[pl.ds(off, 128)], osem.at[0]).start()
        pltpu.make_async_copy(
            rbuf.at[0], wlin_hbm.at[pl.ds(off, 128)], osem.at[0]).wait()

    NTAIL2 = NBLK * 128
    @pl.when(wid == 4)
    def _tail2():
        pltpu.make_async_copy(
            wtail_hbm, rbuf.at[0, pl.ds(0, 64)], isem.at[0]).start()
        pltpu.make_async_copy(
            wtail_hbm, rbuf.at[0, pl.ds(0, 64)], isem.at[0]).wait()
        pltpu.make_async_copy(
            rbuf.at[0, pl.ds(0, 64)], wlin_hbm.at[pl.ds(NTAIL2, 64)],
            osem.at[0]).start()
        pltpu.make_async_copy(
            rbuf.at[0, pl.ds(0, 64)], wlin_hbm.at[pl.ds(NTAIL2, 64)],
            osem.at[0]).wait()


def _gather_body(xb_hbm, wlin_hbm, out_hbm, idx_v, rows_v, tile_v, gsem, ssem):
    wid = lax.axis_index("s") * NC + lax.axis_index("c")
    n = idx_v.shape[0]
    base = wid * n
    pltpu.sync_copy(xb_hbm.at[pl.ds(pl.multiple_of(base, 8), n)], idx_v)
    iota = _iota16()
    zero = iota - iota
    bvecs = [iota + 16 * k for k in range(8)]

    def gather(slot, g):
        return pltpu.make_async_copy(
            wlin_hbm.at[idx_v.at[g]], rows_v.at[slot], gsem.at[slot])

    def stores(slot, g):
        blk = base + g
        t = blk // 32
        bg = lax.rem(blk, 32)
        return [pltpu.make_async_copy(
            tile_v.at[slot, cg], out_hbm.at[t, cg, bg], ssem.at[slot])
            for cg in range(8)]

    def transpose(slot):
        rv = rows_v.at[slot]
        tv = tile_v.at[slot]

        @plsc.parallel_loop(0, 64, unroll=4)
        def _chans(c):
            cg = c // 8
            cs = lax.rem(c, 8)
            cvec = zero + c
            for k in range(8):
                vals = plsc.load_gather(rv, [bvecs[k], cvec])
                tv[cg, cs, pl.ds(16 * k, 16)] = vals

    for par in range(4):
        gather(par, par).start()

    def body(m, _):
        for par in range(4):
            g = 4 * m + par

            @pl.when(m > 0)
            def _drain():
                for d in stores(par, g - 4):
                    d.wait()

            gather(par, g).wait()
            transpose(par)

            @pl.when(m < n // 4 - 1)
            def _next():
                gather(par, g + 4).start()

            for d in stores(par, g):
                d.start()
        return _

    lax.fori_loop(0, n // 4, body, None)
    for par in range(4):
        for d in stores(par, n - 4 + par):
            d.wait()


_relayout = pl.kernel(
    _relayout_body,
    out_type=jax.ShapeDtypeStruct((VOCAB_ROWS, 128), jnp.float32),
    mesh=_mesh,
    scratch_types=[
        pltpu.VMEM((4, 64, 128), jnp.float32),
        pltpu.VMEM((4, 128, 128), jnp.float32),
        pltpu.SemaphoreType.DMA((4,)),
        pltpu.SemaphoreType.DMA((4,)),
    ],
    compiler_params=_params,
)

_gather = pl.kernel(
    _gather_body,
    out_type=jax.ShapeDtypeStruct((200, 8, 32, 8, 128), jnp.float32),
    mesh=_mesh,
    scratch_types=[
        pltpu.VMEM((6400 // NW, CHUNK), jnp.int32),
        pltpu.VMEM((4, CHUNK, 128), jnp.float32),
        pltpu.VMEM((4, 8, 8, 128), jnp.float32),
        pltpu.SemaphoreType.DMA((4,)),
        pltpu.SemaphoreType.DMA((4,)),
    ],
    compiler_params=_params,
)


@jax.jit
def kernel(x, weight):
    xb = x.astype(jnp.int32).T.reshape(6400, 128)
    wtail = jnp.pad(weight[NBLK * 128:], ((0, 0), (0, 64)))
    w_lin = _relayout(weight.T, wtail)
    out5 = _gather(xb, w_lin)
    return jnp.transpose(out5, (2, 4, 0, 1, 3)).reshape(4096, 200, D)


# speedup vs baseline: 1.0019x; 1.0019x over previous
import functools
import jax
import jax.numpy as jnp
from jax import lax
from jax.experimental import pallas as pl
from jax.experimental.pallas import tpu as pltpu
from jax.experimental.pallas import tpu_sc as plsc

VOCAB_ROWS = 1000000
D = 64
NC = 2
NS = 16
NW = NC * NS
NBLK = VOCAB_ROWS // 128
NMAIN = (NBLK // NW) * NW
CHUNK = 128

_params = pltpu.CompilerParams(
    use_tc_tiling_on_sc=True, needs_layout_passes=False,
    disable_bounds_checks=True)
_mesh = plsc.VectorSubcoreMesh(core_axis_name="c", subcore_axis_name="s")


def _iota16():
    return lax.iota(jnp.int32, 16)


def _relayout_body(wt_hbm, wtail_hbm, wlin_hbm, tbuf, rbuf, isem, osem):
    wid = lax.axis_index("s") * NC + lax.axis_index("c")
    iota = _iota16()
    zero = iota - iota
    cvecs = [iota + 16 * j for j in range(4)]

    def load(slot, j):
        return pltpu.make_async_copy(
            wt_hbm.at[pl.ds(0, 64),
                      pl.ds(pl.multiple_of((wid + NW * j) * 128, 128), 128)],
            tbuf.at[slot], isem.at[slot])

    def store(slot, j):
        return pltpu.make_async_copy(
            rbuf.at[slot],
            wlin_hbm.at[pl.ds(pl.multiple_of((wid + NW * j) * 128, 128), 128)],
            osem.at[slot])

    def transpose(slot):
        tb = tbuf.at[slot]
        rb = rbuf.at[slot]

        @plsc.parallel_loop(0, 128, unroll=4)
        def _rows(r):
            rsplat = zero + r
            for j in range(4):
                vals = plsc.load_gather(tb, [cvecs[j], rsplat])
                rb[r, pl.ds(16 * j, 16)] = vals

    NB = NMAIN // NW
    NQ = NB // 4

    for par in range(4):
        load(par, par).start()

    def body(m, _):
        for par in range(4):
            j = 4 * m + par
            load(par, j).wait()

            @pl.when(m > 0)
            def _drain():
                store(par, j - 4).wait()

            transpose(par)

            @pl.when(m < NQ - 1)
            def _next():
                load(par, j + 4).start()

            store(par, j).start()
        return _

    lax.fori_loop(0, NQ, body, None)
    for par in range(4):
        store(par, NB - 4 + par).wait()

    @pl.when(wid < 4)
    def _tail():
        off = pl.multiple_of((NMAIN + wid) * 128, 128)
        pltpu.make_async_copy(
            wt_hbm.at[pl.ds(0, 64), pl.ds(off, 128)], tbuf.at[0],
            isem.at[0]).start()
        pltpu.make_async_copy(
            wt_hbm.at[pl.ds(0, 64), pl.ds(off, 128)], tbuf.at[0],
            isem.at[0]).wait()
        transpose(0)
        pltpu.make_async_copy(
            rbuf.at[0], wlin_hbm.at[pl.ds(off, 128)], osem.at[0]).start()
        pltpu.make_async_copy(
            rbuf.at[0], wlin_hbm.at[pl.ds(off, 128)], osem.at[0]).wait()

    NTAIL2 = NBLK * 128
    @pl.when(wid == 4)
    def _tail2():
        pltpu.make_async_copy(
            wtail_hbm, rbuf.at[0, pl.ds(0, 64)], isem.at[0]).start()
        pltpu.make_async_copy(
            wtail_hbm, rbuf.at[0, pl.ds(0, 64)], isem.at[0]).wait()
        pltpu.make_async_copy(
            rbuf.at[0, pl.ds(0, 64)], wlin_hbm.at[pl.ds(NTAIL2, 64)],
            osem.at[0]).start()
        pltpu.make_async_copy(
            rbuf.at[0, pl.ds(0, 64)], wlin_hbm.at[pl.ds(NTAIL2, 64)],
            osem.at[0]).wait()


def _gather_body(xb_hbm, wlin_hbm, out_hbm, idx_v, rows_v, tile_v, gsem, ssem):
    wid = lax.axis_index("s") * NC + lax.axis_index("c")
    n = idx_v.shape[0]
    base = wid * n
    pltpu.sync_copy(xb_hbm.at[pl.ds(pl.multiple_of(base, 8), n)], idx_v)
    iota = _iota16()
    zero = iota - iota
    bvecs = [iota + 16 * k for k in range(8)]

    def gather(slot, g):
        return pltpu.make_async_copy(
            wlin_hbm.at[idx_v.at[g]], rows_v.at[slot], gsem.at[slot])

    def stores(slot, g):
        blk = base + g
        t = blk // 32
        bg = lax.rem(blk, 32)
        return [pltpu.make_async_copy(
            tile_v.at[slot], out_hbm.at[t, pl.ds(0, 8), bg], ssem.at[slot])]

    def transpose(slot):
        rv = rows_v.at[slot]
        tv = tile_v.at[slot]

        @plsc.parallel_loop(0, 64, unroll=4)
        def _chans(c):
            cg = c // 8
            cs = lax.rem(c, 8)
            cvec = zero + c
            for k in range(8):
                vals = plsc.load_gather(rv, [bvecs[k], cvec])
                tv[cg, cs, pl.ds(16 * k, 16)] = vals

    for par in range(4):
        gather(par, par).start()

    def body(m, _):
        for par in range(4):
            g = 4 * m + par

            @pl.when(m > 0)
            def _drain():
                for d in stores(par, g - 4):
                    d.wait()

            gather(par, g).wait()
            transpose(par)

            @pl.when(m < n // 4 - 1)
            def _next():
                gather(par, g + 4).start()

            for d in stores(par, g):
                d.start()
        return _

    lax.fori_loop(0, n // 4, body, None)
    for par in range(4):
        for d in stores(par, n - 4 + par):
            d.wait()


_relayout = pl.kernel(
    _relayout_body,
    out_type=jax.ShapeDtypeStruct((VOCAB_ROWS, 128), jnp.float32),
    mesh=_mesh,
    scratch_types=[
        pltpu.VMEM((4, 64, 128), jnp.float32),
        pltpu.VMEM((4, 128, 128), jnp.float32),
        pltpu.SemaphoreType.DMA((4,)),
        pltpu.SemaphoreType.DMA((4,)),
    ],
    compiler_params=_params,
)

_gather = pl.kernel(
    _gather_body,
    out_type=jax.ShapeDtypeStruct((200, 8, 32, 8, 128), jnp.float32),
    mesh=_mesh,
    scratch_types=[
        pltpu.VMEM((6400 // NW, CHUNK), jnp.int32),
        pltpu.VMEM((4, CHUNK, 128), jnp.float32),
        pltpu.VMEM((4, 8, 8, 128), jnp.float32),
        pltpu.SemaphoreType.DMA((4,)),
        pltpu.SemaphoreType.DMA((4,)),
    ],
    compiler_params=_params,
)


@jax.jit
def kernel(x, weight):
    xb = x.astype(jnp.int32).T.reshape(6400, 128)
    wtail = jnp.pad(weight[NBLK * 128:], ((0, 0), (0, 64)))
    w_lin = _relayout(weight.T, wtail)
    out5 = _gather(xb, w_lin)
    return jnp.transpose(out5, (2, 4, 0, 1, 3)).reshape(4096, 200, D)


# speedup vs baseline: 1.1828x; 1.1805x over previous
import functools
import jax
import jax.numpy as jnp
from jax import lax
from jax.experimental import pallas as pl
from jax.experimental.pallas import tpu as pltpu
from jax.experimental.pallas import tpu_sc as plsc

D = 64
NC = 2
NS = 16
NW = NC * NS
CHUNK = 512
M = 3
H = 1


def _embedding_body(x_hbm, w_hbm, out_hbm, idx_v, rows_v, gsem, ssem):
    wid = lax.axis_index("s") * NC + lax.axis_index("c")
    n = idx_v.shape[0]
    base = wid * n
    pltpu.sync_copy(x_hbm.at[pl.ds(base, n)], idx_v)

    def gather(slot, chunk):
        return pltpu.make_async_copy(
            w_hbm.at[idx_v.at[chunk]], rows_v.at[slot], gsem.at[slot])

    def store(slot, chunk):
        return pltpu.make_async_copy(
            rows_v.at[slot], out_hbm.at[base + chunk], ssem.at[slot])

    def turn(i, s_ahead, s_cur, ahead_live, drain_live):
        if drain_live:
            store(s_ahead, i + H - M).wait()
        if ahead_live:
            gather(s_ahead, i + H).start()
        gather(s_cur, i).wait()
        store(s_cur, i).start()

    for j in range(H):
        gather(j % M, j).start()

    mid0 = M - H
    mid_n = ((n - H) - mid0) // M * M

    for i in range(mid0):
        turn(i, (i + H) % M, i % M, True, i + H >= M)

    def body(k, _):
        first = mid0 + k * M
        for t in range(M):
            s = (mid0 + t) % M
            turn(first + t, (s + H) % M, s, True, True)
        return _

    lax.fori_loop(0, mid_n // M, body, None)

    for i in range(mid0 + mid_n, n):
        live = i + H < n
        turn(i, (i + H) % M, i % M, live, live and i + H >= M)

    for c in range(n - M, n):
        store(c % M, c).wait()


def _make_call(n_chunks):
    chunks_per_w = n_chunks // NW
    mesh = plsc.VectorSubcoreMesh(core_axis_name="c", subcore_axis_name="s")
    return pl.kernel(
        _embedding_body,
        out_type=jax.ShapeDtypeStruct((n_chunks, CHUNK, D), jnp.float32),
        mesh=mesh,
        scratch_types=[
            pltpu.VMEM((chunks_per_w, CHUNK), jnp.int32),
            pltpu.VMEM((M, CHUNK, D), jnp.float32),
            pltpu.SemaphoreType.DMA((M,)),
            pltpu.SemaphoreType.DMA((M,)),
        ],
        compiler_params=pltpu.CompilerParams(use_tc_tiling_on_sc=False),
    )


@jax.jit
def kernel(x, weight):
    s0, s1 = x.shape
    n = s0 * s1
    assert n % (NW * CHUNK) == 0
    xc = x.astype(jnp.int32).reshape(n // CHUNK, CHUNK)
    out = _make_call(n // CHUNK)(xc, weight)
    return out.reshape(s0, s1, D)
